# baseline (speedup 1.0000x reference)
import functools

import jax
import jax.numpy as jnp
import numpy as np
from jax import lax
from jax.experimental import pallas as pl
from jax.experimental.pallas import tpu as pltpu
from jax.experimental.pallas import tpu_sc as plsc

_N = 10000
_E = 320000
_C = 128
_H = 32
_CO = 32
_M = 10000
_DEG = 32

_BM = 400
_GRID = _M // _BM
_BM1 = 2000
_GRID1 = _M // _BM1

_NC = 2
_NS = 16
_NW = _NC * _NS
_EPW = _E // _NW
_CH = 1000
_NCHUNK = _EPW // _CH


def _tc1a_body(inf4_ref, w1t4_ref, a4_ref):
    k = pl.program_id(0)

    @pl.when(k == 0)
    def _init():
        a4_ref[...] = jnp.zeros_like(a4_ref)

    a4_ref[...] += jnp.dot(inf4_ref[...], w1t4_ref[...],
                           preferred_element_type=jnp.float32)


def _tc1a(inf4, w1t4):
    return pl.pallas_call(
        _tc1a_body,
        grid=(4,),
        in_specs=[
            pl.BlockSpec((_N // 4, _C), lambda k: (0, k)),
            pl.BlockSpec((_C, 4 * _H), lambda k: (k, 0)),
        ],
        out_specs=pl.BlockSpec((_N // 4, 4 * _H), lambda k: (0, 0)),
        out_shape=jax.ShapeDtypeStruct((_N // 4, 4 * _H), jnp.float32),
    )(inf4, w1t4)


def _tc1b_body(outf_ref, w1b4_ref, b14_ref, bmt_ref):
    bmt_ref[...] = jnp.dot(outf_ref[...], w1b4_ref[...],
                           preferred_element_type=jnp.float32) + b14_ref[...]


def _tc1b(out_f, w1b4, b14):
    return pl.pallas_call(
        _tc1b_body,
        grid=(_GRID1,),
        in_specs=[
            pl.BlockSpec((_BM1, _C), lambda i: (i, 0)),
            pl.BlockSpec((_C, 4 * _H), lambda i: (0, 0)),
            pl.BlockSpec((1, 4 * _H), lambda i: (0, 0)),
        ],
        out_specs=pl.BlockSpec((_BM1, 4 * _H), lambda i: (i, 0)),
        out_shape=jax.ShapeDtypeStruct((_M, 4 * _H), jnp.float32),
    )(out_f, w1b4, b14)


def _sc_gather_body(a_hbm, idx_hbm, w_hbm, perm_hbm, g_hbm, wg_hbm,
                    idx_v, perm_v, rows_v, wrow_v, *sems):
    isem = sems[0:2]
    psem = sems[2:4]
    gsem = sems[4:6]
    wsem = sems[6:8]
    ogsem = sems[8:10]
    owsem = sems[10:12]
    wid = lax.axis_index("s") * _NC + lax.axis_index("c")
    ebase = wid * _EPW

    def load_inputs(c, b):
        base = ebase + c * _CH
        cpi = pltpu.async_copy(idx_hbm.at[pl.ds(base, _CH)], idx_v.at[b],
                               isem[b])
        cpp = pltpu.async_copy(perm_hbm.at[pl.ds(base, _CH)], perm_v.at[b],
                               psem[b])
        return cpi, cpp

    def issue_gathers(b):
        cpa = pltpu.async_copy(a_hbm.at[idx_v.at[b]], rows_v.at[b], gsem[b])
        cpw = pltpu.async_copy(w_hbm.at[idx_v.at[b]], wrow_v.at[b], wsem[b])
        return cpa, cpw

    cpi, cpp = load_inputs(0, 0)
    cpi.wait()
    cpp.wait()
    gath = {0: issue_gathers(0)}
    pend_out = {}
    for c in range(_NCHUNK):
        b = c % 2
        nb = 1 - b
        if c + 1 < _NCHUNK:
            cpi, cpp = load_inputs(c + 1, nb)
        cpa, cpw = gath.pop(c)
        cpa.wait()
        cpw.wait()
        base = ebase + c * _CH
        cpo_g = pltpu.async_copy(rows_v.at[b], g_hbm.at[perm_v.at[b]],
                                 ogsem[b])
        cpo_w = pltpu.async_copy(wrow_v.at[b], wg_hbm.at[pl.ds(base, _CH)],
                                 owsem[b])
        if c + 1 < _NCHUNK:
            cpi.wait()
            cpp.wait()
            if nb in pend_out:
                og, ow = pend_out.pop(nb)
                og.wait()
                ow.wait()
            gath[c + 1] = issue_gathers(nb)
        pend_out[b] = (cpo_g, cpo_w)
    for b in list(pend_out):
        og, ow = pend_out.pop(b)
        og.wait()
        ow.wait()


@functools.lru_cache(maxsize=1)
def _sc_gather_kernel():
    return pl.kernel(
        _sc_gather_body,
        mesh=plsc.VectorSubcoreMesh(core_axis_name="c", subcore_axis_name="s"),
        compiler_params=pltpu.CompilerParams(use_tc_tiling_on_sc=False),
        out_type=[
            jax.ShapeDtypeStruct((_E, _H), jnp.float32),
            jax.ShapeDtypeStruct((_E,), jnp.float32),
        ],
        scratch_types=[
            pltpu.VMEM((2, _CH), jnp.int32),
            pltpu.VMEM((2, _CH), jnp.int32),
            pltpu.VMEM((2, _CH, _H), jnp.float32),
            pltpu.VMEM((2, _CH), jnp.float32),
        ] + [pltpu.SemaphoreType.DMA] * 12,
    )


def _gelu_exact(x):
    return 0.5 * x * (1.0 + lax.erf(x * 0.7071067811865476))


def _tc2_body(g_ref, wg_ref, bmt_ref, r_ref, w2s_ref, b2_ref, o_ref, acc_ref):
    s = pl.program_id(1)

    @pl.when(s == 0)
    def _init():
        acc_ref[...] = jnp.zeros((_BM, 4 * _H), jnp.float32)

    xs = g_ref[0] + bmt_ref[...]
    hs = _gelu_exact(xs)
    ws = jnp.dot(wg_ref[...], r_ref[0], preferred_element_type=jnp.float32)
    acc_ref[...] += ws * hs

    @pl.when(s == 7)
    def _fin():
        wsum = jnp.sum(wg_ref[...], axis=1, keepdims=True)
        o_ref[...] = (jnp.dot(acc_ref[...], w2s_ref[...],
                              preferred_element_type=jnp.float32)
                      + wsum * b2_ref[...]) * (1.0 / _DEG)


def _tc2(gs, wg2, bmt, r, w2s, b2row):
    return pl.pallas_call(
        _tc2_body,
        grid=(_GRID, 8),
        in_specs=[
            pl.BlockSpec((1, _BM, 128), lambda i, s: (s, i, 0)),
            pl.BlockSpec((_BM, _DEG), lambda i, s: (i, 0)),
            pl.BlockSpec((_BM, 4 * _H), lambda i, s: (i, 0)),
            pl.BlockSpec((1, _DEG, 128), lambda i, s: (s, 0, 0)),
            pl.BlockSpec((4 * _H, _CO), lambda i, s: (0, 0)),
            pl.BlockSpec((1, _CO), lambda i, s: (0, 0)),
        ],
        out_specs=pl.BlockSpec((_BM, _CO), lambda i, s: (i, 0)),
        out_shape=jax.ShapeDtypeStruct((_M, _CO), jnp.float32),
        scratch_shapes=[pltpu.VMEM((_BM, 4 * _H), jnp.float32)],
    )(gs, wg2, bmt, r, w2s, b2row)


_SS, _JJ = np.meshgrid(np.arange(8), np.arange(128), indexing="ij")
_R_NP = (np.arange(_DEG)[None, :, None] == (_SS * 4 + _JJ // 32)[:, None, :])
_R_CONST = _R_NP.astype(np.float32)

_EIDX = np.arange(_E)
_KK, _MMidx = _EIDX % _DEG, _EIDX // _DEG
_PERM_CONST = ((_KK // 4) * (4 * _M) + _MMidx * 4 + (_KK % 4)).astype(np.int32)


def kernel(in_features, neighbors_0, neighbors_1, out_features, in_weights,
           W1, b1, W2, b2):
    del neighbors_1
    w1t = W1[:_C]
    z = jnp.zeros((_C, _H), jnp.float32)
    w1t4 = jnp.concatenate([
        jnp.concatenate([w1t, z, z, z], axis=1),
        jnp.concatenate([z, w1t, z, z], axis=1),
        jnp.concatenate([z, z, w1t, z], axis=1),
        jnp.concatenate([z, z, z, w1t], axis=1),
    ], axis=0)
    w1b4 = jnp.concatenate([W1[_C:]] * 4, axis=1)
    b14 = jnp.concatenate([b1] * 4).reshape(1, 4 * _H)
    w2s = jnp.concatenate([W2] * 4, axis=0)
    b2row = b2.reshape(1, _CO)
    inf4 = in_features.reshape(_N // 4, 4 * _C)

    a4 = _tc1a(inf4, w1t4)
    bmt = _tc1b(out_features[0], w1b4, b14)
    a = a4.reshape(_N, _H)
    g, wg = _sc_gather_kernel()(a, neighbors_0, in_weights, _PERM_CONST)
    gs = g.reshape(8, _M, 128)
    wg2 = wg.reshape(_M, _DEG)
    out = _tc2(gs, wg2, bmt, _R_CONST, w2s, b2row)
    return out[None]


# speedup vs baseline: 1.6325x; 1.6325x over previous
import functools

import jax
import jax.numpy as jnp
import numpy as np
from jax import lax
from jax.experimental import pallas as pl
from jax.experimental.pallas import tpu as pltpu
from jax.experimental.pallas import tpu_sc as plsc

_N = 10000
_E = 320000
_C = 128
_H = 32
_CO = 32
_M = 10000
_DEG = 32

_BM = 400
_GRID = _M // _BM
_BM1 = 2000
_GRID1 = _M // _BM1

_NC = 2
_NS = 16
_NW = _NC * _NS
_EPW = _E // _NW
_CH = 1000
_NCHUNK = _EPW // _CH


def _tc1a_body(inf4_ref, w1t4_ref, a4_ref):
    k = pl.program_id(0)

    @pl.when(k == 0)
    def _init():
        a4_ref[...] = jnp.zeros_like(a4_ref)

    a4_ref[...] += jnp.dot(inf4_ref[...], w1t4_ref[...],
                           preferred_element_type=jnp.float32)


def _tc1a(inf4, w1t4):
    return pl.pallas_call(
        _tc1a_body,
        grid=(4,),
        in_specs=[
            pl.BlockSpec((_N // 4, _C), lambda k: (0, k)),
            pl.BlockSpec((_C, 4 * _H), lambda k: (k, 0)),
        ],
        out_specs=pl.BlockSpec((_N // 4, 4 * _H), lambda k: (0, 0)),
        out_shape=jax.ShapeDtypeStruct((_N // 4, 4 * _H), jnp.float32),
    )(inf4, w1t4)


def _tc1b_body(outf_ref, w1b4_ref, b14_ref, bmt_ref):
    bmt_ref[...] = jnp.dot(outf_ref[...], w1b4_ref[...],
                           preferred_element_type=jnp.float32) + b14_ref[...]


def _tc1b(out_f, w1b4, b14):
    return pl.pallas_call(
        _tc1b_body,
        grid=(_GRID1,),
        in_specs=[
            pl.BlockSpec((_BM1, _C), lambda i: (i, 0)),
            pl.BlockSpec((_C, 4 * _H), lambda i: (0, 0)),
            pl.BlockSpec((1, 4 * _H), lambda i: (0, 0)),
        ],
        out_specs=pl.BlockSpec((_BM1, 4 * _H), lambda i: (i, 0)),
        out_shape=jax.ShapeDtypeStruct((_M, 4 * _H), jnp.float32),
    )(out_f, w1b4, b14)


def _sc_gather_body(a_hbm, idx_hbm, w_hbm, perm_hbm, g_hbm, wg_hbm,
                    idx_v, perm_v, rows_v, wrow_v, *sems):
    isem = sems[0:2]
    psem = sems[2:4]
    gsem = sems[4:6]
    wsem = sems[6:8]
    ogsem = sems[8:10]
    owsem = sems[10:12]
    wid = lax.axis_index("s") * _NC + lax.axis_index("c")
    ebase = wid * _EPW

    def load_inputs(c, b):
        base = ebase + c * _CH
        cpi = pltpu.async_copy(idx_hbm.at[pl.ds(base, _CH)], idx_v.at[b],
                               isem[b])
        cpp = pltpu.async_copy(perm_hbm.at[pl.ds(base, _CH)], perm_v.at[b],
                               psem[b])
        return cpi, cpp

    def issue_gathers(b):
        cpa = pltpu.async_copy(a_hbm.at[idx_v.at[b]], rows_v.at[b], gsem[b])
        cpw = pltpu.async_copy(w_hbm.at[idx_v.at[b]], wrow_v.at[b], wsem[b])
        return cpa, cpw

    cpi, cpp = load_inputs(0, 0)
    cpi.wait()
    cpp.wait()
    gath = {0: issue_gathers(0)}
    pend_out = {}
    for c in range(_NCHUNK):
        b = c % 2
        nb = 1 - b
        if c + 1 < _NCHUNK:
            cpi, cpp = load_inputs(c + 1, nb)
        cpa, cpw = gath.pop(c)
        cpa.wait()
        cpw.wait()
        base = ebase + c * _CH
        cpo_g = pltpu.async_copy(rows_v.at[b], g_hbm.at[perm_v.at[b]],
                                 ogsem[b])
        cpo_w = pltpu.async_copy(wrow_v.at[b], wg_hbm.at[pl.ds(base, _CH)],
                                 owsem[b])
        if c + 1 < _NCHUNK:
            cpi.wait()
            cpp.wait()
            if nb in pend_out:
                og, ow = pend_out.pop(nb)
                og.wait()
                ow.wait()
            gath[c + 1] = issue_gathers(nb)
        pend_out[b] = (cpo_g, cpo_w)
    for b in list(pend_out):
        og, ow = pend_out.pop(b)
        og.wait()
        ow.wait()


@functools.lru_cache(maxsize=1)
def _sc_gather_kernel():
    return pl.kernel(
        _sc_gather_body,
        mesh=plsc.VectorSubcoreMesh(core_axis_name="c", subcore_axis_name="s"),
        compiler_params=pltpu.CompilerParams(use_tc_tiling_on_sc=False),
        out_type=[
            jax.ShapeDtypeStruct((_E, _H), jnp.float32),
            jax.ShapeDtypeStruct((_E,), jnp.float32),
        ],
        scratch_types=[
            pltpu.VMEM((2, _CH), jnp.int32),
            pltpu.VMEM((2, _CH), jnp.int32),
            pltpu.VMEM((2, _CH, _H), jnp.float32),
            pltpu.VMEM((2, _CH), jnp.float32),
        ] + [pltpu.SemaphoreType.DMA] * 12,
    )


def _gelu_exact(x):
    return 0.5 * x * (1.0 + lax.erf(x * 0.7071067811865476))


def _tc2_body(g_ref, wg_ref, bmt_ref, r_ref, w2s_ref, b2_ref, o_ref):
    g3 = g_ref[...]
    bmt = bmt_ref[...]
    wgb = wg_ref[...]
    acc = jnp.zeros((_BM, 4 * _H), jnp.float32)
    for s in range(8):
        xs = g3[s] + bmt
        hs = _gelu_exact(xs)
        ws = jnp.dot(wgb, r_ref[s], preferred_element_type=jnp.float32)
        acc = acc + ws * hs
    wsum = jnp.sum(wgb, axis=1, keepdims=True)
    o_ref[...] = (jnp.dot(acc, w2s_ref[...],
                          preferred_element_type=jnp.float32)
                  + wsum * b2_ref[...]) * (1.0 / _DEG)


def _tc2(gs, wg2, bmt, r, w2s, b2row):
    return pl.pallas_call(
        _tc2_body,
        grid=(_GRID,),
        in_specs=[
            pl.BlockSpec((8, _BM, 128), lambda i: (0, i, 0)),
            pl.BlockSpec((_BM, _DEG), lambda i: (i, 0)),
            pl.BlockSpec((_BM, 4 * _H), lambda i: (i, 0)),
            pl.BlockSpec((8, _DEG, 128), lambda i: (0, 0, 0)),
            pl.BlockSpec((4 * _H, _CO), lambda i: (0, 0)),
            pl.BlockSpec((1, _CO), lambda i: (0, 0)),
        ],
        out_specs=pl.BlockSpec((_BM, _CO), lambda i: (i, 0)),
        out_shape=jax.ShapeDtypeStruct((_M, _CO), jnp.float32),
    )(gs, wg2, bmt, r, w2s, b2row)


_SS, _JJ = np.meshgrid(np.arange(8), np.arange(128), indexing="ij")
_R_NP = (np.arange(_DEG)[None, :, None] == (_SS * 4 + _JJ // 32)[:, None, :])
_R_CONST = _R_NP.astype(np.float32)

_EIDX = np.arange(_E)
_KK, _MMidx = _EIDX % _DEG, _EIDX // _DEG
_PERM_CONST = ((_KK // 4) * (4 * _M) + _MMidx * 4 + (_KK % 4)).astype(np.int32)


def kernel(in_features, neighbors_0, neighbors_1, out_features, in_weights,
           W1, b1, W2, b2):
    del neighbors_1
    w1t = W1[:_C]
    z = jnp.zeros((_C, _H), jnp.float32)
    w1t4 = jnp.concatenate([
        jnp.concatenate([w1t, z, z, z], axis=1),
        jnp.concatenate([z, w1t, z, z], axis=1),
        jnp.concatenate([z, z, w1t, z], axis=1),
        jnp.concatenate([z, z, z, w1t], axis=1),
    ], axis=0)
    w1b4 = jnp.concatenate([W1[_C:]] * 4, axis=1)
    b14 = jnp.concatenate([b1] * 4).reshape(1, 4 * _H)
    w2s = jnp.concatenate([W2] * 4, axis=0)
    b2row = b2.reshape(1, _CO)
    inf4 = in_features.reshape(_N // 4, 4 * _C)

    a4 = _tc1a(inf4, w1t4)
    bmt = _tc1b(out_features[0], w1b4, b14)
    a = a4.reshape(_N, _H)
    g, wg = _sc_gather_kernel()(a, neighbors_0, in_weights, _PERM_CONST)
    gs = g.reshape(8, _M, 128)
    wg2 = wg.reshape(_M, _DEG)
    out = _tc2(gs, wg2, bmt, _R_CONST, w2s, b2row)
    return out[None]
